# Initial kernel scaffold; baseline (speedup 1.0000x reference)
import jax
import jax.numpy as jnp
from jax.experimental import pallas as pl


def kernel(input_ids, embed_weight):
    raise NotImplementedError("write your pallas kernel here")


# speedup vs baseline: 1.1882x; 1.1882x over previous
import functools

import jax
import jax.numpy as jnp
from jax import lax
from jax.experimental import pallas as pl
from jax.experimental.pallas import tpu as pltpu
from jax.experimental.pallas import tpu_sc as plsc

NTOKEN = 1000000
NINP = 32
BATCH = 16384
SEQ = 50

NC = 2
NS = 16
NW = NC * NS
TOT = BATCH * SEQ
PER_W = TOT // NW
CHUNK = 128
NCHUNK = PER_W // CHUNK


def _emb_body(idx_hbm, table_hbm, out_hbm, idx_v, rows_v, sem):
    wid = lax.axis_index("s") * NC + lax.axis_index("c")
    pltpu.sync_copy(idx_hbm.at[wid], idx_v)

    def step(j, carry):
        pltpu.async_copy(table_hbm.at[idx_v.at[j]], rows_v, sem).wait()
        pltpu.sync_copy(rows_v, out_hbm.at[wid, j])
        return carry

    lax.fori_loop(0, NCHUNK, step, 0)


@jax.jit
def _emb(idx, table):
    mesh = plsc.VectorSubcoreMesh(core_axis_name="c", subcore_axis_name="s")
    k = pl.kernel(
        _emb_body,
        mesh=mesh,
        compiler_params=pltpu.CompilerParams(use_tc_tiling_on_sc=False),
        out_type=jax.ShapeDtypeStruct((NW, NCHUNK, CHUNK, NINP), jnp.float32),
        scratch_types=[
            pltpu.VMEM((NCHUNK, CHUNK), jnp.int32),
            pltpu.VMEM((CHUNK, NINP), jnp.float32),
            pltpu.SemaphoreType.DMA,
        ],
    )
    return k(idx, table)


def kernel(input_ids, embed_weight):
    idx = input_ids.reshape(-1).astype(jnp.int32).reshape(NW, NCHUNK, CHUNK)
    out = _emb(idx, embed_weight)
    return out.reshape(BATCH, SEQ, NINP)


# speedup vs baseline: 1.3062x; 1.0994x over previous
import functools

import jax
import jax.numpy as jnp
from jax import lax
from jax.experimental import pallas as pl
from jax.experimental.pallas import tpu as pltpu
from jax.experimental.pallas import tpu_sc as plsc

NTOKEN = 1000000
NINP = 32
BATCH = 16384
SEQ = 50

NC = 2
NS = 16
NW = NC * NS
TOT = BATCH * SEQ
PER_W = TOT // NW
CHUNK = 128
NCHUNK = PER_W // CHUNK


RING = 10
NGROUP = NCHUNK // RING


def _emb_body(idx_hbm, table_hbm, out_hbm, idx_v, rows_v, gsems, wsems):
    wid = lax.axis_index("s") * NC + lax.axis_index("c")
    pltpu.sync_copy(idx_hbm.at[wid], idx_v)

    def group(i, carry):
        for b in range(RING):
            j = i * RING + b

            @pl.when(i > 0)
            def _():
                pltpu.make_async_copy(rows_v.at[b], out_hbm.at[wid, j], wsems[b]).wait()

            pltpu.make_async_copy(table_hbm.at[idx_v.at[j]], rows_v.at[b], gsems[b]).start()
        for b in range(RING):
            j = i * RING + b
            copy = pltpu.make_async_copy(table_hbm.at[idx_v.at[j]], rows_v.at[b], gsems[b])
            copy.wait()
            pltpu.make_async_copy(rows_v.at[b], out_hbm.at[wid, j], wsems[b]).start()
        return carry

    lax.fori_loop(0, NGROUP, group, 0)
    for b in range(RING):
        j = (NGROUP - 1) * RING + b
        pltpu.make_async_copy(rows_v.at[b], out_hbm.at[wid, j], wsems[b]).wait()


@jax.jit
def _emb(idx, table):
    mesh = plsc.VectorSubcoreMesh(core_axis_name="c", subcore_axis_name="s")
    k = pl.kernel(
        _emb_body,
        mesh=mesh,
        compiler_params=pltpu.CompilerParams(use_tc_tiling_on_sc=False),
        out_type=jax.ShapeDtypeStruct((NW, NCHUNK, CHUNK, NINP), jnp.float32),
        scratch_types=[
            pltpu.VMEM((NCHUNK, CHUNK), jnp.int32),
            pltpu.VMEM((RING, CHUNK, NINP), jnp.float32),
            [pltpu.SemaphoreType.DMA] * RING,
            [pltpu.SemaphoreType.DMA] * RING,
        ],
    )
    return k(idx, table)


def kernel(input_ids, embed_weight):
    idx = input_ids.reshape(-1).astype(jnp.int32).reshape(NW, NCHUNK, CHUNK)
    out = _emb(idx, embed_weight)
    return out.reshape(BATCH, SEQ, NINP)
